# baseline (speedup 1.0000x reference)
import functools

import jax
import jax.numpy as jnp
from jax import lax
from jax.experimental import pallas as pl
from jax.experimental.pallas import tpu as pltpu
from jax.experimental.pallas import tpu_sc as plsc

D = 64
PK = 128
L = 16
CHUNK = 128
PASSES = 2


@functools.lru_cache(maxsize=None)
def _make_sc_kernel(B):
    info = plsc.get_sparse_core_info()
    NC, NS = info.num_cores, info.num_subcores
    NW = NC * NS
    bpw = B // NW
    hrows = bpw // PASSES
    nchunk = hrows // CHUNK
    assert B % (NW * PASSES * CHUNK) == 0

    mesh = plsc.VectorSubcoreMesh(core_axis_name="c", subcore_axis_name="s")

    @functools.partial(
        pl.kernel,
        mesh=mesh,
        compiler_params=pltpu.CompilerParams(needs_layout_passes=False, use_tc_tiling_on_sc=True),
        out_type=jax.ShapeDtypeStruct((B,), jnp.float32),
        scratch_types=[
            pltpu.VMEM((bpw,), jnp.int32),
            pltpu.VMEM((bpw,), jnp.int32),
            pltpu.VMEM((nchunk, CHUNK), jnp.int32),
            pltpu.VMEM((nchunk, CHUNK), jnp.int32),
            pltpu.VMEM((hrows, PK), jnp.float32),
            pltpu.VMEM((hrows, PK), jnp.float32),
            pltpu.VMEM((bpw,), jnp.float32),
            pltpu.VMEM((L,), jnp.float32),
            pltpu.VMEM((L,), jnp.float32),
            pltpu.SemaphoreType.DMA,
        ],
    )
    def sc_kernel(idx_t_hbm, idx_c_hbm, table_hbm, w_hbm, b_hbm, out_hbm,
                  idx_t_v, idx_c_v, pidx_t_v, pidx_c_v, rows_t_v, rows_c_v,
                  out_v, w_v, b_v, sem):
        wid = lax.axis_index("s") * NC + lax.axis_index("c")
        base = wid * bpw

        pltpu.sync_copy(idx_t_hbm.at[wid], idx_t_v)
        pltpu.sync_copy(idx_c_hbm.at[wid], idx_c_v)
        pltpu.sync_copy(w_hbm, w_v)
        pltpu.sync_copy(b_hbm, b_v)

        wv = w_v[...]
        bv = b_v[...]
        lane_iota = lax.iota(jnp.int32, L)

        for p in range(PASSES):
            poff = p * hrows

            def pk_body(i, carry):
                j = i // (CHUNK // L)
                s = pl.ds((i % (CHUNK // L)) * L, L)
                ps = pl.ds(poff + i * L, L)
                pidx_t_v[j, s] = idx_t_v[ps] >> 1
                pidx_c_v[j, s] = idx_c_v[ps] >> 1
                return carry

            lax.fori_loop(0, hrows // L, pk_body, 0)

            copies = []
            for j in range(nchunk):
                copies.append(pltpu.async_copy(
                    table_hbm.at[pidx_t_v.at[j]],
                    rows_t_v.at[pl.ds(j * CHUNK, CHUNK)], sem))
                copies.append(pltpu.async_copy(
                    table_hbm.at[pidx_c_v.at[j]],
                    rows_c_v.at[pl.ds(j * CHUNK, CHUNK)], sem))
            for cp in copies:
                cp.wait()

            def group_body(g, carry):
                s = pl.ds(poff + g * L, L)
                off_t = (idx_t_v[s] & 1) << 6
                off_c = (idx_c_v[s] & 1) << 6
                rows = jnp.full((L,), g * L, jnp.int32) + lane_iota

                def col_body(d, acc):
                    vt = plsc.load_gather(rows_t_v, [rows, off_t + d])
                    vc = plsc.load_gather(rows_c_v, [rows, off_c + d])
                    return acc + vt * vc

                acc = lax.fori_loop(0, D, col_body,
                                    jnp.zeros((L,), jnp.float32))
                z = acc * wv + bv
                out_v[pl.ds(poff + g * L, L)] = 1.0 / (1.0 + jnp.exp(-z))
                return carry

            lax.fori_loop(0, hrows // L, group_body, 0)

        pltpu.sync_copy(out_v, out_hbm.at[pl.ds(base, bpw)])

    return sc_kernel, NW


def kernel(input_target, input_context, embedding_table, dense_w, dense_b):
    B = input_target.shape[0]
    sc_kernel, NW = _make_sc_kernel(B)
    table_pk = embedding_table.reshape(embedding_table.shape[0] // 2, PK)
    idx_t = input_target.reshape(NW, B // NW).astype(jnp.int32)
    idx_c = input_context.reshape(NW, B // NW).astype(jnp.int32)
    w_arr = jnp.full((L,), dense_w[0, 0], jnp.float32)
    b_arr = jnp.full((L,), dense_b[0], jnp.float32)
    out = sc_kernel(idx_t, idx_c, table_pk, w_arr, b_arr)
    return out.reshape(B, 1)
